# Initial kernel scaffold; baseline (speedup 1.0000x reference)
import jax
import jax.numpy as jnp
from jax.experimental import pallas as pl


def kernel(x, adj_rows, adj_cols, adj_vals, W, b):
    raise NotImplementedError("write your pallas kernel here")


# speedup vs baseline: 2.3361x; 2.3361x over previous
import functools

import jax
import jax.numpy as jnp
from jax.experimental import pallas as pl
from jax.experimental.pallas import tpu as pltpu
from jax.experimental.pallas import tpu_sc as plsc

_LANES = 16
_WINDOW = 128


def _dense_body(x_ref, w_ref, b_ref, o_ref):
    o_ref[...] = (
        jnp.dot(x_ref[...], w_ref[...], preferred_element_type=jnp.float32)
        + b_ref[...]
    )


def _dense(x, W, b):
    n, d_in = x.shape
    d_out = W.shape[1]
    blk = 2000
    return pl.pallas_call(
        _dense_body,
        grid=(n // blk,),
        in_specs=[
            pl.BlockSpec((blk, d_in), lambda i: (i, 0)),
            pl.BlockSpec((d_in, d_out), lambda i: (0, 0)),
            pl.BlockSpec((1, d_out), lambda i: (0, 0)),
        ],
        out_specs=pl.BlockSpec((blk, d_out), lambda i: (i, 0)),
        out_shape=jax.ShapeDtypeStruct((n, d_out), jnp.float32),
    )(x, W, b.reshape(1, d_out))


def _edge_combine(z, rows, cols, vals):
    e_total = rows.shape[0]
    d = z.shape[1]
    mesh = plsc.VectorSubcoreMesh(
        core_axis_name="core", subcore_axis_name="subcore"
    )

    @functools.partial(
        pl.kernel,
        out_type=jax.ShapeDtypeStruct((e_total, d), jnp.float32),
        mesh=mesh,
        scratch_types=[
            pltpu.VMEM((_WINDOW, d), jnp.float32),
            pltpu.VMEM((_WINDOW, d), jnp.float32),
            pltpu.SemaphoreType.DMA,
            pltpu.SemaphoreType.DMA,
        ],
    )
    def k(z_hbm, r_hbm, c_hbm, v_hbm, o_hbm, rbuf, cbuf, sem_r, sem_c):
        def body(ri_vmem, ci_vmem, v_vmem, o_vmem):
            cp_r = pltpu.async_copy(z_hbm.at[ri_vmem.at[0]], rbuf, sem_r)
            cp_c = pltpu.async_copy(z_hbm.at[ci_vmem.at[0]], cbuf, sem_c)
            cp_r.wait()
            cp_c.wait()

            @pl.loop(0, _WINDOW, step=_LANES)
            def _(e0):
                vvec = v_vmem[0, pl.ds(e0, _LANES)]
                for j in range(_LANES):
                    val = vvec[j]
                    for g in range(0, d, _LANES):
                        o_vmem[e0 + j, pl.ds(g, _LANES)] = val * (
                            rbuf[e0 + j, pl.ds(g, _LANES)]
                            + cbuf[e0 + j, pl.ds(g, _LANES)]
                        )

        pltpu.emit_pipeline(
            body,
            grid=(e_total // _WINDOW,),
            in_specs=[
                pl.BlockSpec((1, _WINDOW), lambda i: (0, i)),
                pl.BlockSpec((1, _WINDOW), lambda i: (0, i)),
                pl.BlockSpec((1, _WINDOW), lambda i: (0, i)),
            ],
            out_specs=[pl.BlockSpec((_WINDOW, d), lambda i: (i, 0))],
            core_axis_name=("core", "subcore"),
            dimension_semantics=(pltpu.PARALLEL,),
        )(r_hbm, c_hbm, v_hbm, o_hbm)

    return k(
        z,
        rows.reshape(1, e_total),
        cols.reshape(1, e_total),
        vals.reshape(1, e_total),
    )


def kernel(x, adj_rows, adj_cols, adj_vals, W, b):
    z = _dense(x, W, b)
    return _edge_combine(z, adj_rows, adj_cols, adj_vals)


# speedup vs baseline: 2.7238x; 1.1660x over previous
import functools

import jax
import jax.numpy as jnp
from jax import lax
from jax.experimental import pallas as pl
from jax.experimental.pallas import tpu as pltpu
from jax.experimental.pallas import tpu_sc as plsc

_LANES = 16
_WINDOW = 128


def _dense_body(x_ref, w_ref, b_ref, o_ref):
    o_ref[...] = (
        jnp.dot(x_ref[...], w_ref[...], preferred_element_type=jnp.float32)
        + b_ref[...]
    )


def _dense(x, W, b):
    n, d_in = x.shape
    d_out = W.shape[1]
    blk = 2000
    return pl.pallas_call(
        _dense_body,
        grid=(n // blk,),
        in_specs=[
            pl.BlockSpec((blk, d_in), lambda i: (i, 0)),
            pl.BlockSpec((d_in, d_out), lambda i: (0, 0)),
            pl.BlockSpec((1, d_out), lambda i: (0, 0)),
        ],
        out_specs=pl.BlockSpec((blk, d_out), lambda i: (i, 0)),
        out_shape=jax.ShapeDtypeStruct((n, d_out), jnp.float32),
    )(x, W, b.reshape(1, d_out))


def _edge_combine(z, rows, cols, vals):
    e_total = rows.shape[0]
    d = z.shape[1]
    nwin = e_total // _WINDOW
    mesh = plsc.VectorSubcoreMesh(
        core_axis_name="core", subcore_axis_name="subcore"
    )
    n_workers = 32
    max_j = -(-nwin // n_workers)
    if max_j % 2:
        max_j += 1

    @functools.partial(
        pl.kernel,
        out_type=jax.ShapeDtypeStruct((e_total, d), jnp.float32),
        mesh=mesh,
        scratch_types=[
            pltpu.VMEM((_WINDOW,), jnp.int32),
            pltpu.VMEM((_WINDOW,), jnp.int32),
            pltpu.VMEM((_WINDOW,), jnp.float32),
            pltpu.VMEM((_WINDOW, 128), jnp.float32),
            pltpu.VMEM((_WINDOW, 128), jnp.float32),
            pltpu.VMEM((_WINDOW, 128), jnp.float32),
            pltpu.VMEM((_WINDOW,), jnp.int32),
            pltpu.VMEM((_WINDOW,), jnp.int32),
            pltpu.VMEM((_WINDOW,), jnp.float32),
            pltpu.VMEM((_WINDOW, 128), jnp.float32),
            pltpu.VMEM((_WINDOW, 128), jnp.float32),
            pltpu.VMEM((_WINDOW, 128), jnp.float32),
            pltpu.SemaphoreType.DMA,
            pltpu.SemaphoreType.DMA,
            pltpu.SemaphoreType.DMA,
            pltpu.SemaphoreType.DMA,
            pltpu.SemaphoreType.DMA,
            pltpu.SemaphoreType.DMA,
        ],
    )
    def k(z_hbm, r_hbm, c_hbm, v_hbm, o_hbm, *scratch):
        slots = (
            dict(
                ri=scratch[0], ci=scratch[1], vv=scratch[2],
                rb=scratch[3], cb=scratch[4], ob=scratch[5],
                sem_i=scratch[12], sem_g=scratch[13], sem_o=scratch[14],
            ),
            dict(
                ri=scratch[6], ci=scratch[7], vv=scratch[8],
                rb=scratch[9], cb=scratch[10], ob=scratch[11],
                sem_i=scratch[15], sem_g=scratch[16], sem_o=scratch[17],
            ),
        )
        t = lax.axis_index("subcore") * 2 + lax.axis_index("core")

        def win(j):
            return t + n_workers * j

        def issue_idx(j, s):
            @pl.when(win(j) < nwin)
            def _():
                w = win(j)
                pltpu.async_copy(r_hbm.at[w], s["ri"], s["sem_i"])
                pltpu.async_copy(c_hbm.at[w], s["ci"], s["sem_i"])
                pltpu.async_copy(v_hbm.at[w], s["vv"], s["sem_i"])

        def issue_gather(j, s):
            @pl.when(win(j) < nwin)
            def _():
                pltpu.make_async_copy(r_hbm.at[0], s["ri"], s["sem_i"]).wait()
                pltpu.make_async_copy(c_hbm.at[0], s["ci"], s["sem_i"]).wait()
                pltpu.make_async_copy(v_hbm.at[0], s["vv"], s["sem_i"]).wait()
                pltpu.async_copy(z_hbm.at[s["ri"]], s["rb"], s["sem_g"])
                pltpu.async_copy(z_hbm.at[s["ci"]], s["cb"], s["sem_g"])

        def half(j, s):
            @pl.when(win(j) < nwin)
            def _():
                w = win(j)
                pltpu.make_async_copy(
                    z_hbm.at[s["ri"]], s["rb"], s["sem_g"]
                ).wait()
                pltpu.make_async_copy(
                    z_hbm.at[s["ci"]], s["cb"], s["sem_g"]
                ).wait()

                @pl.when(j >= 2)
                def _():
                    pltpu.make_async_copy(
                        s["ob"], o_hbm.at[pl.ds(0, _WINDOW)], s["sem_o"]
                    ).wait()

                rb, cb, ob, vv = s["rb"], s["cb"], s["ob"], s["vv"]

                @pl.loop(0, _WINDOW, step=_LANES)
                def _(e0):
                    vvec = vv[pl.ds(e0, _LANES)]
                    for u in range(_LANES):
                        val = vvec[u]
                        for g in range(0, d, _LANES):
                            ob[e0 + u, pl.ds(g, _LANES)] = val * (
                                rb[e0 + u, pl.ds(g, _LANES)]
                                + cb[e0 + u, pl.ds(g, _LANES)]
                            )

                pltpu.async_copy(
                    s["ob"], o_hbm.at[pl.ds(w * _WINDOW, _WINDOW)], s["sem_o"]
                )
                issue_idx(j + 2, s)
                issue_gather(j + 2, s)

        issue_idx(0, slots[0])
        issue_idx(1, slots[1])
        issue_gather(0, slots[0])
        issue_gather(1, slots[1])

        @pl.loop(0, max_j, step=2)
        def _(jj):
            half(jj, slots[0])
            half(jj + 1, slots[1])

        for s in slots:
            pltpu.make_async_copy(
                o_hbm.at[pl.ds(0, _WINDOW)], s["ob"], s["sem_o"]
            ).wait()

    return k(
        z,
        rows.reshape(nwin, _WINDOW),
        cols.reshape(nwin, _WINDOW),
        vals.reshape(nwin, _WINDOW),
    )


def kernel(x, adj_rows, adj_cols, adj_vals, W, b):
    z = _dense(x, W, b)
    return _edge_combine(z, adj_rows, adj_cols, adj_vals)


# speedup vs baseline: 3.9960x; 1.4670x over previous
import dataclasses
import functools

import jax
import jax.numpy as jnp
from jax import lax
from jax.experimental import pallas as pl
from jax.experimental.pallas import tpu as pltpu
from jax.experimental.pallas import tpu_sc as plsc

_LANES = 16
_WINDOW = 128


def _dense_body(x_ref, w_ref, b_ref, o_ref):
    o_ref[...] = (
        jnp.dot(x_ref[...], w_ref[...], preferred_element_type=jnp.float32)
        + b_ref[...]
    ).astype(jnp.bfloat16)


def _dense_bf16(x, W, b):
    n, d_in = x.shape
    d_out = W.shape[1]
    blk = 2000
    return pl.pallas_call(
        _dense_body,
        grid=(n // blk,),
        in_specs=[
            pl.BlockSpec((blk, d_in), lambda i: (i, 0)),
            pl.BlockSpec((d_in, d_out), lambda i: (0, 0)),
            pl.BlockSpec((1, d_out), lambda i: (0, 0)),
        ],
        out_specs=pl.BlockSpec((blk, d_out), lambda i: (i, 0)),
        out_shape=jax.ShapeDtypeStruct((n, d_out), jnp.bfloat16),
    )(x, W, b.reshape(1, d_out))


def _edge_combine(z_packed, idx_combined, e_total, d):
    nwin = e_total // _WINDOW
    dw = d // 2
    mesh = plsc.VectorSubcoreMesh(
        core_axis_name="core", subcore_axis_name="subcore"
    )
    n_workers = 32
    max_j = -(-nwin // n_workers)
    if max_j % 2:
        max_j += 1

    cp = pltpu.CompilerParams()
    if "needs_layout_passes" in pltpu.CompilerParams.__dataclass_fields__:
        cp = dataclasses.replace(cp, needs_layout_passes=False)
    if "use_tc_tiling_on_sc" in pltpu.CompilerParams.__dataclass_fields__:
        cp = dataclasses.replace(cp, use_tc_tiling_on_sc=False)

    @functools.partial(
        pl.kernel,
        out_type=jax.ShapeDtypeStruct((e_total, d), jnp.float32),
        mesh=mesh,
        compiler_params=cp,
        scratch_types=[
            pltpu.VMEM((3 * _WINDOW,), jnp.int32),
            pltpu.VMEM((_WINDOW, dw), jnp.int32),
            pltpu.VMEM((_WINDOW, dw), jnp.int32),
            pltpu.VMEM((_WINDOW, d), jnp.float32),
            pltpu.VMEM((3 * _WINDOW,), jnp.int32),
            pltpu.VMEM((_WINDOW, dw), jnp.int32),
            pltpu.VMEM((_WINDOW, dw), jnp.int32),
            pltpu.VMEM((_WINDOW, d), jnp.float32),
            pltpu.SemaphoreType.DMA,
            pltpu.SemaphoreType.DMA,
            pltpu.SemaphoreType.DMA,
            pltpu.SemaphoreType.DMA,
            pltpu.SemaphoreType.DMA,
            pltpu.SemaphoreType.DMA,
        ],
    )
    def k(z_hbm, i_hbm, o_hbm, *scratch):
        slots = (
            dict(
                ib=scratch[0], rb=scratch[1], cb=scratch[2], ob=scratch[3],
                sem_i=scratch[8], sem_g=scratch[9], sem_o=scratch[10],
            ),
            dict(
                ib=scratch[4], rb=scratch[5], cb=scratch[6], ob=scratch[7],
                sem_i=scratch[11], sem_g=scratch[12], sem_o=scratch[13],
            ),
        )
        t = lax.axis_index("subcore") * 2 + lax.axis_index("core")

        def win(j):
            return t + n_workers * j

        def issue_idx(j, s):
            @pl.when(win(j) < nwin)
            def _():
                pltpu.async_copy(i_hbm.at[win(j)], s["ib"], s["sem_i"])

        def issue_gather(j, s):
            @pl.when(win(j) < nwin)
            def _():
                pltpu.make_async_copy(i_hbm.at[0], s["ib"], s["sem_i"]).wait()
                pltpu.async_copy(
                    z_hbm.at[s["ib"].at[pl.ds(0, _WINDOW)]], s["rb"], s["sem_g"]
                )
                pltpu.async_copy(
                    z_hbm.at[s["ib"].at[pl.ds(_WINDOW, _WINDOW)]],
                    s["cb"],
                    s["sem_g"],
                )

        himask = jnp.int32(-65536)

        def half(j, s):
            @pl.when(win(j) < nwin)
            def _():
                w = win(j)
                pltpu.make_async_copy(
                    z_hbm.at[s["ib"].at[pl.ds(0, _WINDOW)]], s["rb"], s["sem_g"]
                ).wait()
                pltpu.make_async_copy(
                    z_hbm.at[s["ib"].at[pl.ds(_WINDOW, _WINDOW)]],
                    s["cb"],
                    s["sem_g"],
                ).wait()

                @pl.when(j >= 2)
                def _():
                    pltpu.make_async_copy(
                        s["ob"], o_hbm.at[pl.ds(0, _WINDOW)], s["sem_o"]
                    ).wait()

                ib, rb, cb, ob = s["ib"], s["rb"], s["cb"], s["ob"]

                @pl.loop(0, _WINDOW, step=_LANES)
                def _(e0):
                    vvec = plsc.bitcast(
                        ib[pl.ds(2 * _WINDOW + e0, _LANES)], jnp.float32
                    )
                    for u in range(_LANES):
                        val = vvec[u]
                        for g in range(0, dw, _LANES):
                            wr = rb[e0 + u, pl.ds(g, _LANES)]
                            wc = cb[e0 + u, pl.ds(g, _LANES)]
                            s_lo = plsc.bitcast(
                                wr << 16, jnp.float32
                            ) + plsc.bitcast(wc << 16, jnp.float32)
                            s_hi = plsc.bitcast(
                                wr & himask, jnp.float32
                            ) + plsc.bitcast(wc & himask, jnp.float32)
                            ob[e0 + u, pl.ds(g, _LANES)] = val * s_lo
                            ob[e0 + u, pl.ds(dw + g, _LANES)] = val * s_hi

                pltpu.async_copy(
                    s["ob"], o_hbm.at[pl.ds(w * _WINDOW, _WINDOW)], s["sem_o"]
                )
                issue_idx(j + 2, s)
                issue_gather(j + 2, s)

        issue_idx(0, slots[0])
        issue_idx(1, slots[1])
        issue_gather(0, slots[0])
        issue_gather(1, slots[1])

        @pl.loop(0, max_j, step=2)
        def _(jj):
            half(jj, slots[0])
            half(jj + 1, slots[1])

        for s in slots:
            pltpu.make_async_copy(
                o_hbm.at[pl.ds(0, _WINDOW)], s["ob"], s["sem_o"]
            ).wait()

    return k(z_packed, idx_combined)


def kernel(x, adj_rows, adj_cols, adj_vals, W, b):
    n, d_in = x.shape
    d = W.shape[1]
    e_total = adj_rows.shape[0]
    nwin = e_total // _WINDOW
    perm = jnp.arange(d).reshape(2, d // 2).T.reshape(-1)
    z_bf16 = _dense_bf16(x, W[:, perm], b[perm])
    z_packed = lax.bitcast_convert_type(
        z_bf16.reshape(n, d // 2, 2), jnp.int32
    )
    idx_combined = jnp.concatenate(
        [
            adj_rows.reshape(nwin, _WINDOW),
            adj_cols.reshape(nwin, _WINDOW),
            lax.bitcast_convert_type(adj_vals, jnp.int32).reshape(
                nwin, _WINDOW
            ),
        ],
        axis=1,
    )
    return _edge_combine(z_packed, idx_combined, e_total, d)
